# Initial kernel scaffold; baseline (speedup 1.0000x reference)
import jax
import jax.numpy as jnp
from jax.experimental import pallas as pl


def kernel(query, value, embeddings):
    raise NotImplementedError("write your pallas kernel here")


# speedup vs baseline: 7.6209x; 7.6209x over previous
import functools

import jax
import jax.numpy as jnp
from jax import lax
from jax.experimental import pallas as pl
from jax.experimental.pallas import tpu as pltpu
from jax.experimental.pallas import tpu_sc as plsc

L_Q = 2048
L_V = 2048
N_EMB = 129
D = 64
MAXP = (N_EMB - 1) // 2
FILL_LO = L_V - 1 - MAXP
HB = L_Q
WROWS = L_V * D // 128

NC = 2
NS = 16
NW = NC * NS
CHUNK = 2 * HB // NS
ROWS_PER_W = L_Q // NW


def _sc_band_kernel(emb_hbm, out_hbm, table_v, stage_e, stage_o, b_even, b_odd):
    c = lax.axis_index("c")
    s = lax.axis_index("s")

    pltpu.sync_copy(emb_hbm, table_v)

    base = s * CHUNK

    def build_row(r2, _):
        k = base + 2 * r2
        t0 = jnp.clip(k - FILL_LO, 0, N_EMB - 1)
        t1 = jnp.clip(k + 1 - FILL_LO, 0, N_EMB - 1)
        t2 = jnp.clip(k + 2 - FILL_LO, 0, N_EMB - 1)
        for col in range(D // 16):
            sl = pl.ds(col * 16, 16)
            sh = pl.ds(D + col * 16, 16)
            stage_e[r2, sl] = table_v[t0, sl]
            stage_e[r2, sh] = table_v[t1, sl]
            stage_o[r2, sl] = table_v[t1, sl]
            stage_o[r2, sh] = table_v[t2, sl]
        return _

    lax.fori_loop(0, CHUNK // 2, build_row, 0)
    pltpu.sync_copy(stage_e, b_even.at[pl.ds(s * (CHUNK // 2), CHUNK // 2)])
    pltpu.sync_copy(stage_o, b_odd.at[pl.ds(s * (CHUNK // 2), CHUNK // 2)])
    plsc.subcore_barrier()

    wid = s * NC + c
    i0 = wid * ROWS_PER_W

    def do_even(rr, _):
        i = i0 + 2 * rr
        pltpu.sync_copy(b_odd.at[pl.ds((L_V - 2 - i) // 2, WROWS)], out_hbm.at[i])
        return _

    def do_odd(rr, _):
        i = i0 + 2 * rr + 1
        pltpu.sync_copy(b_even.at[pl.ds((L_V - 1 - i) // 2, WROWS)], out_hbm.at[i])
        return _

    lax.fori_loop(0, ROWS_PER_W // 2, do_even, 0)
    lax.fori_loop(0, ROWS_PER_W // 2, do_odd, 0)


def kernel(query, value, embeddings):
    del query, value
    mesh = plsc.VectorSubcoreMesh(core_axis_name="c", subcore_axis_name="s")
    f = functools.partial(
        pl.kernel,
        mesh=mesh,
        out_type=jax.ShapeDtypeStruct((L_Q, WROWS, 128), jnp.float32),
        scratch_types=[
            pltpu.VMEM((N_EMB, D), jnp.float32),
            pltpu.VMEM((CHUNK // 2, 128), jnp.float32),
            pltpu.VMEM((CHUNK // 2, 128), jnp.float32),
            pltpu.VMEM_SHARED((HB, 128), jnp.float32),
            pltpu.VMEM_SHARED((HB, 128), jnp.float32),
        ],
    )(_sc_band_kernel)
    return f(embeddings).reshape(L_Q, L_V, D)
